# Initial kernel scaffold; baseline (speedup 1.0000x reference)
import jax
import jax.numpy as jnp
from jax.experimental import pallas as pl


def kernel(hidden_states, sequence_mask, W_down_fused, W_up_fused, key_perm, Wo):
    raise NotImplementedError("write your pallas kernel here")


# speedup vs baseline: 5.1834x; 5.1834x over previous
import math

import jax
import jax.numpy as jnp
from jax import lax
from jax.experimental import pallas as pl
from jax.experimental.pallas import tpu as pltpu

_VMEM_LIMIT = 60 * 1024 * 1024


def _collapse_kernel(a_ref, b_ref, o_ref):
    o_ref[...] = jnp.dot(a_ref[...].astype(jnp.bfloat16),
                         b_ref[...].astype(jnp.bfloat16),
                         preferred_element_type=jnp.float32).astype(o_ref.dtype)


def _collapse(a, b):
    K, N = b.shape
    M = a.shape[0]
    tm = M // 2
    return pl.pallas_call(
        _collapse_kernel,
        out_shape=jax.ShapeDtypeStruct((M, N), jnp.bfloat16),
        grid=(2,),
        in_specs=[pl.BlockSpec((tm, K), lambda i: (i, 0)),
                  pl.BlockSpec((K, N), lambda i: (0, 0))],
        out_specs=pl.BlockSpec((tm, N), lambda i: (i, 0)),
        compiler_params=pltpu.CompilerParams(
            dimension_semantics=("parallel",),
            vmem_limit_bytes=_VMEM_LIMIT),
    )(a, b)


def _proj_rope_kernel(x_ref, w_ref, cos_ref, sin_ref, q_ref, k_ref, v_ref):
    nq = q_ref.shape[1]
    nkv = k_ref.shape[1]
    y = jnp.dot(x_ref[...].astype(jnp.bfloat16), w_ref[...],
                preferred_element_type=jnp.float32)
    tm = y.shape[0]
    nqk = (nq + nkv) * 128
    qk = y[:, :nqk].reshape(tm, nq + nkv, 128)
    cc = cos_ref[...][:, None, :]
    ss = sin_ref[...][:, None, :]
    ab = qk[..., :64]
    swapped = jnp.concatenate([ab[..., 32:], ab[..., :32]], axis=-1)
    rot = jnp.concatenate([ab * cc + swapped * ss, qk[..., 64:]], axis=-1)
    q_ref[...] = rot[:, :nq, :].astype(q_ref.dtype)
    k_ref[...] = rot[:, nq:, :].astype(k_ref.dtype)
    v_ref[...] = y[:, nqk:].reshape(tm, nkv, 128).astype(v_ref.dtype)


def _proj_rope(x, w, cos, sin, nq, nkv, tm):
    M, d_model = x.shape
    grid = (M // tm,)
    q, k, v = pl.pallas_call(
        _proj_rope_kernel,
        out_shape=(jax.ShapeDtypeStruct((M, nq, 128), jnp.bfloat16),
                   jax.ShapeDtypeStruct((M, nkv, 128), jnp.bfloat16),
                   jax.ShapeDtypeStruct((M, nkv, 128), jnp.bfloat16)),
        grid=grid,
        in_specs=[pl.BlockSpec((tm, d_model), lambda i: (i, 0)),
                  pl.BlockSpec(w.shape, lambda i: (0, 0)),
                  pl.BlockSpec((tm, 64), lambda i: (i, 0)),
                  pl.BlockSpec((tm, 64), lambda i: (i, 0))],
        out_specs=(pl.BlockSpec((tm, nq, 128), lambda i: (i, 0, 0)),
                   pl.BlockSpec((tm, nkv, 128), lambda i: (i, 0, 0)),
                   pl.BlockSpec((tm, nkv, 128), lambda i: (i, 0, 0))),
        compiler_params=pltpu.CompilerParams(
            dimension_semantics=("parallel",),
            vmem_limit_bytes=_VMEM_LIMIT),
    )(x, w, cos, sin)
    return q, k, v


def _attn_out_kernel(q_ref, k_ref, v_ref, wo_ref, o_ref):
    qi = pl.program_id(1)
    tq, nq = q_ref.shape[0], q_ref.shape[2]
    nkv = k_ref.shape[2]
    n_rep = nq // nkv
    Dv = v_ref.shape[-1]
    R = tq * n_rep
    S = k_ref.shape[0]
    q3 = q_ref[:, 0]

    def compute(nk):
        k3 = k_ref[:nk, 0]
        v3 = v_ref[:nk, 0]
        qpos = qi * tq + lax.broadcasted_iota(jnp.int32, (R, nk), 0) // n_rep
        kpos = lax.broadcasted_iota(jnp.int32, (R, nk), 1)
        causal = kpos <= qpos
        acc = None
        for h in range(nkv):
            qf = q3[:, h * n_rep:(h + 1) * n_rep, :].reshape(R, 128)
            kk = k3[:, h, :]
            scores = lax.dot_general(qf, kk, (((1,), (1,)), ((), ())),
                                     preferred_element_type=jnp.float32)
            scores = jnp.where(causal, scores, -1e30)
            m = jnp.max(scores, axis=-1, keepdims=True)
            p = jnp.exp(scores - m)
            l = jnp.sum(p, axis=-1, keepdims=True)
            pv = jnp.dot(p.astype(jnp.bfloat16), v3[:, h, :],
                         preferred_element_type=jnp.float32)
            ob = (pv * pl.reciprocal(l, approx=True)
                  ).reshape(tq, n_rep * Dv).astype(jnp.bfloat16)
            wo_blk = wo_ref[h * n_rep * Dv:(h + 1) * n_rep * Dv, :]
            d = jnp.dot(ob, wo_blk, preferred_element_type=jnp.float32)
            acc = d if acc is None else acc + d
        o_ref[...] = acc.reshape(tq, nq, Dv)[:, None]

    if S == tq:
        compute(S)
    else:
        for i in range(S // tq):
            @pl.when(qi == i)
            def _(nk=(i + 1) * tq):
                compute(nk)


def _attn_out(q4, k4, v4, wo, tq):
    S, B, nq, D = q4.shape
    nkv = k4.shape[2]
    d_model = wo.shape[1]
    out = pl.pallas_call(
        _attn_out_kernel,
        out_shape=jax.ShapeDtypeStruct((S, B, nq, D), jnp.float32),
        grid=(B, S // tq),
        in_specs=[
            pl.BlockSpec((tq, 1, nq, D), lambda b, qi: (qi, b, 0, 0)),
            pl.BlockSpec((S, 1, nkv, D), lambda b, qi: (0, b, 0, 0)),
            pl.BlockSpec((S, 1, nkv, D), lambda b, qi: (0, b, 0, 0)),
            pl.BlockSpec(wo.shape, lambda b, qi: (0, 0)),
        ],
        out_specs=pl.BlockSpec((tq, 1, nq, D), lambda b, qi: (qi, b, 0, 0)),
        compiler_params=pltpu.CompilerParams(
            dimension_semantics=("parallel", "parallel"),
            vmem_limit_bytes=_VMEM_LIMIT),
    )(q4, k4, v4, wo)
    return out.reshape(S, B, d_model)


def kernel(hidden_states, sequence_mask, W_down_fused, W_up_fused, key_perm, Wo):
    S, B, d_model = hidden_states.shape
    NQ, NKV, D = 16, 4, 128
    NREP = NQ // NKV
    nq_cols = NQ * D
    low = W_up_fused.shape[0]
    n_rope = W_down_fused.shape[1] - nq_cols - low
    n_nope = NKV * D - n_rope
    rh = n_rope // NKV
    topk = rh // 2
    scale = 1.0 / math.sqrt(D)
    M = S * B

    x = hidden_states.reshape(M, d_model)

    half = D // 2
    Wq4 = W_down_fused[:, :nq_cols].reshape(d_model, NQ, D)
    Wq4 = jnp.concatenate([Wq4[:, :, :topk], Wq4[:, :, half:half + topk],
                           Wq4[:, :, topk:half], Wq4[:, :, half + topk:]],
                          axis=-1)
    Wq = (Wq4 * scale).reshape(d_model, nq_cols).astype(jnp.bfloat16)
    Wkr = W_down_fused[:, nq_cols:nq_cols + n_rope].astype(jnp.bfloat16)
    Wdc = W_down_fused[:, nq_cols + n_rope:]

    Weff = _collapse(Wdc, W_up_fused)
    Wkc = Weff[:, :n_nope]
    Wv = Weff[:, n_nope:]
    Wk = jnp.concatenate([Wkr.reshape(d_model, NKV, rh),
                          Wkc.reshape(d_model, NKV, rh)],
                         axis=-1).reshape(d_model, NKV * D)
    W_a = jnp.concatenate([Wq, Wk, Wv], axis=1)

    pos = jnp.arange(S, dtype=jnp.float32)
    inv_freq = 1.0 / (10000.0 ** (
        jnp.arange(0, 2 * topk, 2, dtype=jnp.float32) / D))
    freqs = pos[:, None] * inv_freq[None, :]
    c32, s32 = jnp.cos(freqs), jnp.sin(freqs)
    cos = jnp.repeat(jnp.concatenate([c32, c32], axis=1), B, axis=0)
    sin = jnp.repeat(jnp.concatenate([-s32, s32], axis=1), B, axis=0)

    q, k, v = _proj_rope(x, W_a, cos, sin, NQ, NKV, tm=min(256, M))
    q4 = q.reshape(S, B, NQ, D)
    k4 = k.reshape(S, B, NKV, D)
    v4 = v.reshape(S, B, NKV, D)

    out = _attn_out(q4, k4, v4, Wo.astype(jnp.bfloat16), tq=min(256, S))
    return {"hidden_states": out, "sequence_mask": sequence_mask}


# speedup vs baseline: 8.0149x; 1.5463x over previous
import functools
import math

import jax
import jax.numpy as jnp
from jax import lax
from jax.experimental import pallas as pl
from jax.experimental.pallas import tpu as pltpu

_VMEM_LIMIT = 60 * 1024 * 1024


def _wprep_kernel(wd_ref, wup_ref, wo_ref, wa_ref, wob_ref, *, cfg):
    nq, nkv, D, topk, scale = cfg
    half = D // 2
    rh = 2 * topk
    nq_cols = nq * D
    n_rope = nkv * rh
    tm = wd_ref.shape[0]

    wq = wd_ref[:, :nq_cols].reshape(tm, nq, D)
    wq = jnp.concatenate([wq[:, :, :topk], wq[:, :, half:half + topk],
                          wq[:, :, topk:half], wq[:, :, half + topk:]],
                         axis=-1) * scale
    wa_ref[:, :nq_cols] = wq.reshape(tm, nq_cols).astype(jnp.bfloat16)

    weff = jnp.dot(wd_ref[:, nq_cols + n_rope:].astype(jnp.bfloat16),
                   wup_ref[...].astype(jnp.bfloat16),
                   preferred_element_type=jnp.float32)
    n_nope = nkv * D - n_rope
    kr = wd_ref[:, nq_cols:nq_cols + n_rope].reshape(tm, nkv, rh)
    kc = weff[:, :n_nope].reshape(tm, nkv, D - rh)
    wk = jnp.concatenate([kr, kc], axis=-1).reshape(tm, nkv * D)
    wa_ref[:, nq_cols:nq_cols + nkv * D] = wk.astype(jnp.bfloat16)
    wa_ref[:, nq_cols + nkv * D:] = weff[:, n_nope:].astype(jnp.bfloat16)

    wob_ref[...] = wo_ref[...].astype(jnp.bfloat16)


def _weight_prep(w_down, w_up, wo, nq, nkv, D, topk, scale):
    d_model = w_down.shape[0]
    n_tot = (nq + 2 * nkv) * D
    tm = d_model // 8
    kfn = functools.partial(_wprep_kernel, cfg=(nq, nkv, D, topk, scale))
    return pl.pallas_call(
        kfn,
        out_shape=(jax.ShapeDtypeStruct((d_model, n_tot), jnp.bfloat16),
                   jax.ShapeDtypeStruct(wo.shape, jnp.bfloat16)),
        grid=(8,),
        in_specs=[pl.BlockSpec((tm, w_down.shape[1]), lambda i: (i, 0)),
                  pl.BlockSpec(w_up.shape, lambda i: (0, 0)),
                  pl.BlockSpec((tm, wo.shape[1]), lambda i: (i, 0))],
        out_specs=(pl.BlockSpec((tm, n_tot), lambda i: (i, 0)),
                   pl.BlockSpec((tm, wo.shape[1]), lambda i: (i, 0))),
        compiler_params=pltpu.CompilerParams(
            dimension_semantics=("parallel",),
            vmem_limit_bytes=_VMEM_LIMIT),
    )(w_down, w_up, wo)


def _proj_rope_kernel(x_ref, w_ref, cos_ref, sin_ref, q_ref, k_ref, v_ref):
    nq = q_ref.shape[1]
    nkv = k_ref.shape[1]
    y = jnp.dot(x_ref[...].astype(jnp.bfloat16), w_ref[...],
                preferred_element_type=jnp.float32)
    tm = y.shape[0]
    nqk = (nq + nkv) * 128
    qk = y[:, :nqk].reshape(tm, nq + nkv, 128)
    cc = cos_ref[...][:, None, :]
    ss = sin_ref[...][:, None, :]
    ab = qk[..., :64]
    swapped = jnp.concatenate([ab[..., 32:], ab[..., :32]], axis=-1)
    rot = jnp.concatenate([ab * cc + swapped * ss, qk[..., 64:]], axis=-1)
    q_ref[...] = rot[:, :nq, :].astype(q_ref.dtype)
    k_ref[...] = rot[:, nq:, :].astype(k_ref.dtype)
    v_ref[...] = y[:, nqk:].reshape(tm, nkv, 128).astype(v_ref.dtype)


def _proj_rope(x, w, cos, sin, nq, nkv, tm):
    M, d_model = x.shape
    grid = (M // tm,)
    q, k, v = pl.pallas_call(
        _proj_rope_kernel,
        out_shape=(jax.ShapeDtypeStruct((M, nq, 128), jnp.bfloat16),
                   jax.ShapeDtypeStruct((M, nkv, 128), jnp.bfloat16),
                   jax.ShapeDtypeStruct((M, nkv, 128), jnp.bfloat16)),
        grid=grid,
        in_specs=[pl.BlockSpec((tm, d_model), lambda i: (i, 0)),
                  pl.BlockSpec(w.shape, lambda i: (0, 0)),
                  pl.BlockSpec((tm, 64), lambda i: (i, 0)),
                  pl.BlockSpec((tm, 64), lambda i: (i, 0))],
        out_specs=(pl.BlockSpec((tm, nq, 128), lambda i: (i, 0, 0)),
                   pl.BlockSpec((tm, nkv, 128), lambda i: (i, 0, 0)),
                   pl.BlockSpec((tm, nkv, 128), lambda i: (i, 0, 0))),
        compiler_params=pltpu.CompilerParams(
            dimension_semantics=("parallel",),
            vmem_limit_bytes=_VMEM_LIMIT),
    )(x, w, cos, sin)
    return q, k, v


def _attn_out_kernel(q_ref, k_ref, v_ref, wo_ref, o_ref):
    qi = pl.program_id(1)
    tq, nq = q_ref.shape[0], q_ref.shape[2]
    nkv = k_ref.shape[2]
    n_rep = nq // nkv
    Dv = v_ref.shape[-1]
    R = tq * n_rep
    S = k_ref.shape[0]
    q3 = q_ref[:, 0]

    def compute(nk):
        k3 = k_ref[:nk, 0]
        v3 = v_ref[:nk, 0]
        qpos = qi * tq + lax.broadcasted_iota(jnp.int32, (R, nk), 0) // n_rep
        kpos = lax.broadcasted_iota(jnp.int32, (R, nk), 1)
        causal = kpos <= qpos
        obs = []
        for h in range(nkv):
            qf = q3[:, h * n_rep:(h + 1) * n_rep, :].reshape(R, 128)
            kk = k3[:, h, :]
            scores = lax.dot_general(qf, kk, (((1,), (1,)), ((), ())),
                                     preferred_element_type=jnp.float32)
            scores = jnp.where(causal, scores, -1e30)
            m = jnp.max(scores, axis=-1, keepdims=True)
            p = jnp.exp(scores - m)
            l = jnp.sum(p, axis=-1, keepdims=True)
            pv = jnp.dot(p.astype(jnp.bfloat16), v3[:, h, :],
                         preferred_element_type=jnp.float32)
            obs.append((pv * pl.reciprocal(l, approx=True)
                        ).reshape(tq, n_rep * Dv).astype(jnp.bfloat16))
        obcat = jnp.concatenate(obs, axis=1)
        o_ref[...] = jnp.dot(obcat, wo_ref[...],
                             preferred_element_type=jnp.float32)

    if S == tq:
        compute(S)
    else:
        for i in range(S // tq):
            @pl.when(qi == i)
            def _(nk=(i + 1) * tq):
                compute(nk)


def _attn_out(q4, k4, v4, wo, tq):
    S, B, nq, D = q4.shape
    nkv = k4.shape[2]
    d_model = wo.shape[1]
    out = pl.pallas_call(
        _attn_out_kernel,
        out_shape=jax.ShapeDtypeStruct((S, B * d_model), jnp.float32),
        grid=(B, S // tq),
        in_specs=[
            pl.BlockSpec((tq, 1, nq, D), lambda b, qi: (qi, b, 0, 0)),
            pl.BlockSpec((S, 1, nkv, D), lambda b, qi: (0, b, 0, 0)),
            pl.BlockSpec((S, 1, nkv, D), lambda b, qi: (0, b, 0, 0)),
            pl.BlockSpec(wo.shape, lambda b, qi: (0, 0)),
        ],
        out_specs=pl.BlockSpec((tq, d_model), lambda b, qi: (qi, b)),
        compiler_params=pltpu.CompilerParams(
            dimension_semantics=("parallel", "parallel"),
            vmem_limit_bytes=_VMEM_LIMIT),
    )(q4, k4, v4, wo)
    return out.reshape(S, B, d_model)


def kernel(hidden_states, sequence_mask, W_down_fused, W_up_fused, key_perm, Wo):
    S, B, d_model = hidden_states.shape
    NQ, NKV, D = 16, 4, 128
    NREP = NQ // NKV
    nq_cols = NQ * D
    low = W_up_fused.shape[0]
    n_rope = W_down_fused.shape[1] - nq_cols - low
    n_nope = NKV * D - n_rope
    rh = n_rope // NKV
    topk = rh // 2
    scale = 1.0 / math.sqrt(D)
    M = S * B

    x = hidden_states.reshape(M, d_model)

    W_a, Wo_b = _weight_prep(W_down_fused, W_up_fused, Wo,
                             NQ, NKV, D, topk, scale)

    pos = jnp.arange(S, dtype=jnp.float32)
    inv_freq = 1.0 / (10000.0 ** (
        jnp.arange(0, 2 * topk, 2, dtype=jnp.float32) / D))
    freqs = pos[:, None] * inv_freq[None, :]
    c32, s32 = jnp.cos(freqs), jnp.sin(freqs)
    cos = jnp.repeat(jnp.concatenate([c32, c32], axis=1), B, axis=0)
    sin = jnp.repeat(jnp.concatenate([-s32, s32], axis=1), B, axis=0)

    q, k, v = _proj_rope(x, W_a, cos, sin, NQ, NKV, tm=min(512, M))
    q4 = q.reshape(S, B, NQ, D)
    k4 = k.reshape(S, B, NKV, D)
    v4 = v.reshape(S, B, NKV, D)

    out = _attn_out(q4, k4, v4, Wo_b, tq=min(256, S))
    return {"hidden_states": out, "sequence_mask": sequence_mask}


# speedup vs baseline: 8.3459x; 1.0413x over previous
import functools
import math

import jax
import jax.numpy as jnp
from jax import lax
from jax.experimental import pallas as pl
from jax.experimental.pallas import tpu as pltpu

_VMEM_LIMIT = 36 * 1024 * 1024


def _wprep_kernel(wd_ref, wup_ref, wo_ref, wa_ref, wob_ref, *, cfg):
    nq, nkv, D, topk, scale = cfg
    half = D // 2
    rh = 2 * topk
    nq_cols = nq * D
    n_rope = nkv * rh
    tm = wd_ref.shape[0]

    wq = wd_ref[:, :nq_cols].reshape(tm, nq, D)
    wq = jnp.concatenate([wq[:, :, :topk], wq[:, :, half:half + topk],
                          wq[:, :, topk:half], wq[:, :, half + topk:]],
                         axis=-1) * scale
    wa_ref[:, :nq_cols] = wq.reshape(tm, nq_cols).astype(jnp.bfloat16)

    weff = jnp.dot(wd_ref[:, nq_cols + n_rope:].astype(jnp.bfloat16),
                   wup_ref[...].astype(jnp.bfloat16),
                   preferred_element_type=jnp.float32)
    n_nope = nkv * D - n_rope
    kr = wd_ref[:, nq_cols:nq_cols + n_rope].reshape(tm, nkv, rh)
    kc = weff[:, :n_nope].reshape(tm, nkv, D - rh)
    wk = jnp.concatenate([kr, kc], axis=-1).reshape(tm, nkv * D)
    wa_ref[:, nq_cols:nq_cols + nkv * D] = wk.astype(jnp.bfloat16)
    wa_ref[:, nq_cols + nkv * D:] = weff[:, n_nope:].astype(jnp.bfloat16)

    wob_ref[...] = wo_ref[...].astype(jnp.bfloat16)


def _weight_prep(w_down, w_up, wo, nq, nkv, D, topk, scale):
    d_model = w_down.shape[0]
    n_tot = (nq + 2 * nkv) * D
    tm = d_model // 8
    kfn = functools.partial(_wprep_kernel, cfg=(nq, nkv, D, topk, scale))
    return pl.pallas_call(
        kfn,
        out_shape=(jax.ShapeDtypeStruct((d_model, n_tot), jnp.bfloat16),
                   jax.ShapeDtypeStruct(wo.shape, jnp.bfloat16)),
        grid=(8,),
        in_specs=[pl.BlockSpec((tm, w_down.shape[1]), lambda i: (i, 0)),
                  pl.BlockSpec(w_up.shape, lambda i: (0, 0)),
                  pl.BlockSpec((tm, wo.shape[1]), lambda i: (i, 0))],
        out_specs=(pl.BlockSpec((tm, n_tot), lambda i: (i, 0)),
                   pl.BlockSpec((tm, wo.shape[1]), lambda i: (i, 0))),
        compiler_params=pltpu.CompilerParams(
            dimension_semantics=("parallel",),
            vmem_limit_bytes=_VMEM_LIMIT),
    )(w_down, w_up, wo)


def _proj_rope_kernel(x_ref, w_ref, cos_ref, sin_ref, q_ref, k_ref, v_ref):
    nq = q_ref.shape[1]
    nkv = k_ref.shape[1]
    y = jnp.dot(x_ref[...].astype(jnp.bfloat16), w_ref[...],
                preferred_element_type=jnp.float32)
    tm = y.shape[0]
    nqk = (nq + nkv) * 128
    qk = y[:, :nqk].reshape(tm, nq + nkv, 128)
    cc = cos_ref[...][:, None, :]
    ss = sin_ref[...][:, None, :]
    ab = qk[..., :64]
    swapped = jnp.concatenate([ab[..., 32:], ab[..., :32]], axis=-1)
    rot = jnp.concatenate([ab * cc + swapped * ss, qk[..., 64:]], axis=-1)
    q_ref[...] = rot[:, :nq, :].astype(q_ref.dtype)
    k_ref[...] = rot[:, nq:, :].astype(k_ref.dtype)
    v_ref[...] = y[:, nqk:].reshape(tm, nkv, 128).astype(v_ref.dtype)


def _proj_rope(x, w, cos, sin, nq, nkv, tm):
    M, d_model = x.shape
    grid = (M // tm,)
    q, k, v = pl.pallas_call(
        _proj_rope_kernel,
        out_shape=(jax.ShapeDtypeStruct((M, nq, 128), jnp.bfloat16),
                   jax.ShapeDtypeStruct((M, nkv, 128), jnp.bfloat16),
                   jax.ShapeDtypeStruct((M, nkv, 128), jnp.bfloat16)),
        grid=grid,
        in_specs=[pl.BlockSpec((tm, d_model), lambda i: (i, 0)),
                  pl.BlockSpec(w.shape, lambda i: (0, 0)),
                  pl.BlockSpec((tm, 64), lambda i: (i, 0)),
                  pl.BlockSpec((tm, 64), lambda i: (i, 0))],
        out_specs=(pl.BlockSpec((tm, nq, 128), lambda i: (i, 0, 0)),
                   pl.BlockSpec((tm, nkv, 128), lambda i: (i, 0, 0)),
                   pl.BlockSpec((tm, nkv, 128), lambda i: (i, 0, 0))),
        compiler_params=pltpu.CompilerParams(
            dimension_semantics=("parallel",),
            vmem_limit_bytes=_VMEM_LIMIT),
    )(x, w, cos, sin)
    return q, k, v


def _attn_out_kernel(q_ref, k_ref, v_ref, wo_ref, o_ref):
    qi = pl.program_id(1)
    tq, nq = q_ref.shape[0], q_ref.shape[2]
    nkv = k_ref.shape[2]
    n_rep = nq // nkv
    Dv = v_ref.shape[-1]
    R = tq * n_rep
    S = k_ref.shape[0]
    q3 = q_ref[:, 0]

    def compute(nk):
        k3 = k_ref[:nk, 0]
        v3 = v_ref[:nk, 0]
        qpos = qi * tq + lax.broadcasted_iota(jnp.int32, (R, nk), 0) // n_rep
        kpos = lax.broadcasted_iota(jnp.int32, (R, nk), 1)
        causal = kpos <= qpos
        obs = []
        for h in range(nkv):
            qf = q3[:, h * n_rep:(h + 1) * n_rep, :].reshape(R, 128)
            kk = k3[:, h, :]
            scores = lax.dot_general(qf, kk, (((1,), (1,)), ((), ())),
                                     preferred_element_type=jnp.float32)
            scores = jnp.where(causal, scores, -1e30)
            m = jnp.max(scores, axis=-1, keepdims=True)
            p = jnp.exp(scores - m)
            l = jnp.sum(p, axis=-1, keepdims=True)
            pv = jnp.dot(p.astype(jnp.bfloat16), v3[:, h, :],
                         preferred_element_type=jnp.float32)
            obs.append((pv * pl.reciprocal(l, approx=True)
                        ).reshape(tq, n_rep * Dv).astype(jnp.bfloat16))
        obcat = jnp.concatenate(obs, axis=1)
        o_ref[...] = jnp.dot(obcat, wo_ref[...],
                             preferred_element_type=jnp.float32)

    if S == tq:
        compute(S)
    else:
        for i in range(S // tq):
            @pl.when(qi == i)
            def _(nk=(i + 1) * tq):
                compute(nk)


def _attn_out(q4, k4, v4, wo, tq):
    S, B, nq, D = q4.shape
    nkv = k4.shape[2]
    d_model = wo.shape[1]
    out = pl.pallas_call(
        _attn_out_kernel,
        out_shape=jax.ShapeDtypeStruct((S, B * d_model), jnp.float32),
        grid=(B, S // tq),
        in_specs=[
            pl.BlockSpec((tq, 1, nq, D), lambda b, qi: (qi, b, 0, 0)),
            pl.BlockSpec((S, 1, nkv, D), lambda b, qi: (0, b, 0, 0)),
            pl.BlockSpec((S, 1, nkv, D), lambda b, qi: (0, b, 0, 0)),
            pl.BlockSpec(wo.shape, lambda b, qi: (0, 0)),
        ],
        out_specs=pl.BlockSpec((tq, d_model), lambda b, qi: (qi, b)),
        compiler_params=pltpu.CompilerParams(
            dimension_semantics=("parallel", "parallel"),
            vmem_limit_bytes=_VMEM_LIMIT),
    )(q4, k4, v4, wo)
    return out.reshape(S, B, d_model)


def kernel(hidden_states, sequence_mask, W_down_fused, W_up_fused, key_perm, Wo):
    S, B, d_model = hidden_states.shape
    NQ, NKV, D = 16, 4, 128
    NREP = NQ // NKV
    nq_cols = NQ * D
    low = W_up_fused.shape[0]
    n_rope = W_down_fused.shape[1] - nq_cols - low
    n_nope = NKV * D - n_rope
    rh = n_rope // NKV
    topk = rh // 2
    scale = 1.0 / math.sqrt(D)
    M = S * B

    x = hidden_states.reshape(M, d_model)

    W_a, Wo_b = _weight_prep(W_down_fused, W_up_fused, Wo,
                             NQ, NKV, D, topk, scale)

    pos = jnp.arange(S, dtype=jnp.float32)
    inv_freq = 1.0 / (10000.0 ** (
        jnp.arange(0, 2 * topk, 2, dtype=jnp.float32) / D))
    freqs = pos[:, None] * inv_freq[None, :]
    c32, s32 = jnp.cos(freqs), jnp.sin(freqs)
    cos = jnp.repeat(jnp.concatenate([c32, c32], axis=1), B, axis=0)
    sin = jnp.repeat(jnp.concatenate([-s32, s32], axis=1), B, axis=0)

    q, k, v = _proj_rope(x, W_a, cos, sin, NQ, NKV, tm=min(512, M))
    q4 = q.reshape(S, B, NQ, D)
    k4 = k.reshape(S, B, NKV, D)
    v4 = v.reshape(S, B, NKV, D)

    out = _attn_out(q4, k4, v4, Wo_b, tq=min(256, S))
    return {"hidden_states": out, "sequence_mask": sequence_mask}
